# Initial kernel scaffold; baseline (speedup 1.0000x reference)
import jax
import jax.numpy as jnp
from jax.experimental import pallas as pl


def kernel(topics, structure_abstracts, embedding):
    raise NotImplementedError("write your pallas kernel here")


# speedup vs baseline: 1.0305x; 1.0305x over previous
import functools

import jax
import jax.numpy as jnp
from jax import lax
from jax.experimental import pallas as pl
from jax.experimental.pallas import tpu as pltpu
from jax.experimental.pallas import tpu_sc as plsc

_B = 4096
_L = 200
_CTX = 32
_N = _B * (_L + 1)
_NC = 2
_NS = 16
_NW = _NC * _NS
_ROWS_PER_W = _N // _NW
_CHUNK = 536
_NCHUNK = _ROWS_PER_W // _CHUNK
_U = 4


def _tanh16(x):
    e = jnp.exp(x + x)
    return 1.0 - 2.0 / (e + 1.0)


def _body(idx_hbm, table_hbm, out_hbm, idx_v, rows_v, sem):
    wid = lax.axis_index("s") * _NC + lax.axis_index("c")
    w_base = wid * _ROWS_PER_W

    def chunk_iter(j, carry):
        base = w_base + j * _CHUNK
        pltpu.sync_copy(idx_hbm.at[pl.ds(base, _CHUNK)], idx_v)
        pltpu.async_copy(table_hbm.at[idx_v], rows_v, sem).wait()

        def row_iter(i, c):
            r0 = i * _U
            for u in range(_U):
                for h in range(2):
                    sl = (r0 + u, pl.ds(16 * h, 16))
                    rows_v[sl] = _tanh16(rows_v[sl])
            return c

        lax.fori_loop(0, _CHUNK // _U, row_iter, 0)
        pltpu.sync_copy(rows_v, out_hbm.at[pl.ds(base, _CHUNK)])
        return carry

    lax.fori_loop(0, _NCHUNK, chunk_iter, 0)


_mesh = plsc.VectorSubcoreMesh(core_axis_name="c", subcore_axis_name="s")

_gather_tanh = functools.partial(
    pl.kernel,
    out_type=jax.ShapeDtypeStruct((_N, _CTX), jnp.float32),
    mesh=_mesh,
    scratch_types=[
        pltpu.VMEM((_CHUNK,), jnp.int32),
        pltpu.VMEM((_CHUNK, _CTX), jnp.float32),
        pltpu.SemaphoreType.DMA,
    ],
    compiler_params=pltpu.CompilerParams(use_tc_tiling_on_sc=False),
)(_body)


def kernel(topics, structure_abstracts, embedding):
    idx = jnp.concatenate(
        [topics.reshape(-1), structure_abstracts.reshape(-1)]
    ).astype(jnp.int32)
    out = _gather_tanh(idx, embedding)
    out1 = out[:_B].reshape(_B, 1, _CTX)
    out2 = out[_B:].reshape(_B, _L, _CTX)
    return (out1, out2)
